# Initial kernel scaffold; baseline (speedup 1.0000x reference)
import jax
import jax.numpy as jnp
from jax.experimental import pallas as pl


def kernel(Xi, Xv, tables, cross_w, cross_b, W1, b1, W2, b2, Wl, bl):
    raise NotImplementedError("write your pallas kernel here")


# speedup vs baseline: 8.5505x; 8.5505x over previous
import functools

import jax
import jax.numpy as jnp
from jax import lax
from jax.experimental import pallas as pl
from jax.experimental.pallas import tpu as pltpu
from jax.experimental.pallas import tpu_sc as plsc

B, F_, V, D = 4096, 26, 1000, 128
H1, H2 = 1024, 1024
CROSS_DEPTH = 3
FD = F_ * D

_NC, _NS = 2, 16
_NW = _NC * _NS
_ROWS = B * F_
_BPW = _ROWS // _NW
_CH = 128
_NCH = _BPW // _CH


def _gather_body(tab, idx, out, idx_v, rows_v, sem):
    wid = lax.axis_index("s") * _NC + lax.axis_index("c")
    base = wid * _NCH
    pltpu.sync_copy(idx.at[wid], idx_v)

    def chunk(ci, carry):
        pltpu.async_copy(tab.at[idx_v.at[ci]], rows_v, sem).wait()
        off = pl.multiple_of((base + ci) * _CH, _CH)
        pltpu.sync_copy(rows_v, out.at[pl.ds(off, _CH)])
        return carry

    lax.fori_loop(0, _NCH, chunk, 0)


def _sc_gather(tables_flat, gidx2d):
    mesh = plsc.VectorSubcoreMesh(core_axis_name="c", subcore_axis_name="s")
    k = functools.partial(
        pl.kernel,
        mesh=mesh,
        out_type=jax.ShapeDtypeStruct((_ROWS, D), jnp.float32),
        scratch_types=[
            pltpu.VMEM((_NCH, _CH), jnp.int32),
            pltpu.VMEM((_CH, D), jnp.float32),
            pltpu.SemaphoreType.DMA,
        ],
    )(_gather_body)
    return k(tables_flat, gidx2d)


def _tc_body(emb_ref, xv_ref, e_ref, cw_ref, cb_ref, w1_ref, b1_ref,
             w2_ref, b2_ref, wlt_ref, wlb_ref, bl_ref, out_ref):
    xv_wide = jnp.dot(xv_ref[...], e_ref[...],
                      preferred_element_type=jnp.float32,
                      precision=lax.Precision.HIGHEST)
    x0 = emb_ref[...] * xv_wide
    xl = x0
    for i in range(CROSS_DEPTH):
        s = jnp.sum(x0 * xl, axis=1, keepdims=True)
        xl = s * cw_ref[i:i + 1, :] + cb_ref[i:i + 1, :] + xl
    h = jnp.dot(x0, w1_ref[...], preferred_element_type=jnp.float32)
    h = jnp.maximum(h + b1_ref[...], 0.0)
    h = jnp.dot(h, w2_ref[...], preferred_element_type=jnp.float32)
    h = jnp.maximum(h + b2_ref[...], 0.0)
    out = (jnp.sum(xl * wlt_ref[...], axis=1, keepdims=True)
           + jnp.sum(h * wlb_ref[...], axis=1, keepdims=True)
           + bl_ref[...])
    out_ref[...] = out


def _tc_dcn(emb2d, Xv, E, cross_w, cross_b, W1, b1, W2, b2, wlt, wlb, bl2,
            bm=256, interpret=False):
    nblk = B // bm
    full = lambda shape: pl.BlockSpec(shape, lambda i: (0, 0))
    out = pl.pallas_call(
        _tc_body,
        grid=(nblk,),
        in_specs=[
            pl.BlockSpec((bm, FD), lambda i: (i, 0)),
            pl.BlockSpec((bm, F_), lambda i: (i, 0)),
            full((F_, FD)),
            full((CROSS_DEPTH, FD)),
            full((CROSS_DEPTH, FD)),
            full((FD, H1)),
            full((1, H1)),
            full((H1, H2)),
            full((1, H2)),
            full((1, FD)),
            full((1, H2)),
            full((1, 1)),
        ],
        out_specs=pl.BlockSpec((bm, 1), lambda i: (i, 0)),
        out_shape=jax.ShapeDtypeStruct((B, 1), jnp.float32),
        compiler_params=pltpu.CompilerParams(
            dimension_semantics=("arbitrary",),
        ),
        interpret=interpret,
    )(emb2d, Xv, E, cross_w, cross_b, W1, b1, W2, b2, wlt, wlb, bl2)
    return out[:, 0]


def kernel(Xi, Xv, tables, cross_w, cross_b, W1, b1, W2, b2, Wl, bl):
    tables_flat = tables.reshape(F_ * V, D)
    gidx = (Xi[:, :, 0].astype(jnp.int32)
            + (jnp.arange(F_, dtype=jnp.int32) * V)[None, :])
    gidx2d = gidx.reshape(_NW, _NCH, _CH)

    emb = _sc_gather(tables_flat, gidx2d)
    emb2d = emb.reshape(B, FD)

    E = jnp.kron(jnp.eye(F_, dtype=jnp.float32),
                 jnp.ones((1, D), dtype=jnp.float32))
    wlt = Wl[:FD, 0][None, :]
    wlb = Wl[FD:, 0][None, :]
    bl2 = bl.reshape(1, 1)
    return _tc_dcn(emb2d, Xv, E, cross_w, cross_b, W1, b1.reshape(1, H1),
                   W2, b2.reshape(1, H2), wlt, wlb, bl2)


# speedup vs baseline: 10.5198x; 1.2303x over previous
import functools

import jax
import jax.numpy as jnp
from jax import lax
from jax.experimental import pallas as pl
from jax.experimental.pallas import tpu as pltpu
from jax.experimental.pallas import tpu_sc as plsc

B, F_, V, D = 4096, 26, 1000, 128
H1, H2 = 1024, 1024
CROSS_DEPTH = 3
FD = F_ * D

_NC, _NS = 2, 16
_NW = _NC * _NS
_ROWS = B * F_
_BPW = _ROWS // _NW
_CH = 128
_NCH = _BPW // _CH


def _gather_body(tab, idx, out, idx_v, rows_v, sem):
    wid = lax.axis_index("s") * _NC + lax.axis_index("c")
    base = wid * _NCH
    pltpu.sync_copy(idx.at[wid], idx_v)

    def chunk(ci, carry):
        pltpu.async_copy(tab.at[idx_v.at[ci]], rows_v, sem).wait()
        off = pl.multiple_of((base + ci) * _CH, _CH)
        pltpu.sync_copy(rows_v, out.at[pl.ds(off, _CH)])
        return carry

    lax.fori_loop(0, _NCH, chunk, 0)


def _sc_gather(tables_flat, gidx2d):
    mesh = plsc.VectorSubcoreMesh(core_axis_name="c", subcore_axis_name="s")
    k = functools.partial(
        pl.kernel,
        mesh=mesh,
        out_type=jax.ShapeDtypeStruct((_ROWS, D), jnp.float32),
        scratch_types=[
            pltpu.VMEM((_NCH, _CH), jnp.int32),
            pltpu.VMEM((_CH, D), jnp.float32),
            pltpu.SemaphoreType.DMA,
        ],
    )(_gather_body)
    return k(tables_flat, gidx2d)


def _tc_body(emb_ref, xv_ref, e_ref, p_ref, uc_ref, w1_ref, b1_ref,
             w2_ref, b2_ref, wlb_ref, bl_ref, out_ref):
    xv = xv_ref[...]
    xv_hi = xv.astype(jnp.bfloat16)
    xv_lo = (xv - xv_hi.astype(jnp.float32)).astype(jnp.bfloat16)
    e = e_ref[...]
    xv_wide = (jnp.dot(xv_hi, e, preferred_element_type=jnp.float32)
               + jnp.dot(xv_lo, e, preferred_element_type=jnp.float32))
    x0 = emb_ref[...] * xv_wide
    a = jnp.sum(x0 * x0, axis=1, keepdims=True)
    t = jnp.dot(x0, p_ref[...], preferred_element_type=jnp.float32)
    p0 = t[:, 0:1]
    p1 = t[:, 1:2]
    q0 = t[:, 2:3]
    q1 = t[:, 3:4]
    r0 = t[:, 4:5]
    s0 = a
    s1 = s0 * p0 + q0 + a
    s2 = s1 * p1 + q1 + s1
    uc = uc_ref[...]
    cross = (r0 + s0 * uc[:, 0:1] + s1 * uc[:, 1:2] + s2 * uc[:, 2:3]
             + uc[:, 3:4])
    h = jnp.dot(x0, w1_ref[...], preferred_element_type=jnp.float32)
    h = jnp.maximum(h + b1_ref[...], 0.0)
    h = jnp.dot(h, w2_ref[...], preferred_element_type=jnp.float32)
    h = jnp.maximum(h + b2_ref[...], 0.0)
    out_ref[...] = (cross
                    + jnp.sum(h * wlb_ref[...], axis=1, keepdims=True)
                    + bl_ref[...])


def _tc_dcn(emb2d, Xv, E, P, uc, W1, b1, W2, b2, wlb, bl2,
            bm=256, interpret=False):
    nblk = B // bm
    full = lambda shape: pl.BlockSpec(shape, lambda i: (0, 0))
    out = pl.pallas_call(
        _tc_body,
        grid=(nblk,),
        in_specs=[
            pl.BlockSpec((bm, FD), lambda i: (i, 0)),
            pl.BlockSpec((bm, F_), lambda i: (i, 0)),
            full((F_, FD)),
            full((FD, 8)),
            full((1, 8)),
            full((FD, H1)),
            full((1, H1)),
            full((H1, H2)),
            full((1, H2)),
            full((1, H2)),
            full((1, 1)),
        ],
        out_specs=pl.BlockSpec((bm, 1), lambda i: (i, 0)),
        out_shape=jax.ShapeDtypeStruct((B, 1), jnp.float32),
        compiler_params=pltpu.CompilerParams(
            dimension_semantics=("arbitrary",),
        ),
        interpret=interpret,
    )(emb2d, Xv, E, P, uc, W1, b1, W2, b2, wlb, bl2)
    return out[:, 0]


def kernel(Xi, Xv, tables, cross_w, cross_b, W1, b1, W2, b2, Wl, bl):
    tables_flat = tables.reshape(F_ * V, D)
    gidx = (Xi[:, :, 0].astype(jnp.int32)
            + (jnp.arange(F_, dtype=jnp.int32) * V)[None, :])
    gidx2d = gidx.reshape(_NW, _NCH, _CH)

    emb = _sc_gather(tables_flat, gidx2d)
    emb2d = emb.reshape(B, FD)

    E = jnp.kron(jnp.eye(F_, dtype=jnp.bfloat16),
                 jnp.ones((1, D), dtype=jnp.bfloat16))
    wlt = Wl[:FD, 0]
    wlb = Wl[FD:, 0][None, :]
    bl2 = bl.reshape(1, 1)
    P = jnp.stack([cross_w[0], cross_w[1], cross_b[0], cross_b[1], wlt,
                   jnp.zeros_like(wlt), jnp.zeros_like(wlt),
                   jnp.zeros_like(wlt)], axis=1)
    u = jnp.sum(cross_w * wlt[None, :], axis=1)
    c = jnp.sum(cross_b * wlt[None, :])
    uc = jnp.concatenate([u, c[None], jnp.zeros((4,), jnp.float32)]
                         ).reshape(1, 8)
    return _tc_dcn(emb2d, Xv, E, P, uc, W1, b1.reshape(1, H1),
                   W2, b2.reshape(1, H2), wlb, bl2)


# speedup vs baseline: 11.3720x; 1.0810x over previous
import functools

import jax
import jax.numpy as jnp
from jax import lax
from jax.experimental import pallas as pl
from jax.experimental.pallas import tpu as pltpu
from jax.experimental.pallas import tpu_sc as plsc

B, F_, V, D = 4096, 26, 1000, 128
H1, H2 = 1024, 1024
CROSS_DEPTH = 3
FD = F_ * D

_NC, _NS = 2, 16
_NW = _NC * _NS
_ROWS = B * F_
_BPW = _ROWS // _NW
_CH = 128
_NCH = _BPW // _CH


def _gather_body(tab, idx, out, idx_v, rows0, rows1, sem0, sem1):
    wid = lax.axis_index("s") * _NC + lax.axis_index("c")
    base = wid * _NCH
    pltpu.sync_copy(idx.at[wid], idx_v)

    def _start(ci, buf, sem):
        pltpu.async_copy(tab.at[idx_v.at[ci]], buf, sem)

    def _drain(ci, buf, sem):
        pltpu.make_async_copy(tab.at[idx_v.at[ci]], buf, sem).wait()
        off = pl.multiple_of((base + ci) * _CH, _CH)
        pltpu.sync_copy(buf, out.at[pl.ds(off, _CH)])

    _start(0, rows0, sem0)
    _start(1, rows1, sem1)

    def step(i, carry):
        _drain(2 * i, rows0, sem0)
        _start(2 * i + 2, rows0, sem0)
        _drain(2 * i + 1, rows1, sem1)
        _start(2 * i + 3, rows1, sem1)
        return carry

    lax.fori_loop(0, _NCH // 2 - 1, step, 0)
    _drain(_NCH - 2, rows0, sem0)
    _drain(_NCH - 1, rows1, sem1)


def _sc_gather(tables_flat, gidx2d):
    mesh = plsc.VectorSubcoreMesh(core_axis_name="c", subcore_axis_name="s")
    k = functools.partial(
        pl.kernel,
        mesh=mesh,
        out_type=jax.ShapeDtypeStruct((_ROWS, D), jnp.float32),
        scratch_types=[
            pltpu.VMEM((_NCH, _CH), jnp.int32),
            pltpu.VMEM((_CH, D), jnp.float32),
            pltpu.VMEM((_CH, D), jnp.float32),
            pltpu.SemaphoreType.DMA,
            pltpu.SemaphoreType.DMA,
        ],
    )(_gather_body)
    return k(tables_flat, gidx2d)


def _tc_body(emb_ref, xv_ref, e_ref, p_ref, uc_ref, w1_ref, b1_ref,
             w2_ref, b2_ref, wlb_ref, bl_ref, out_ref):
    xv = xv_ref[...]
    xv_hi = xv.astype(jnp.bfloat16)
    xv_lo = (xv - xv_hi.astype(jnp.float32)).astype(jnp.bfloat16)
    e = e_ref[...]
    xv_wide = (jnp.dot(xv_hi, e, preferred_element_type=jnp.float32)
               + jnp.dot(xv_lo, e, preferred_element_type=jnp.float32))
    x0 = emb_ref[...] * xv_wide
    a = jnp.sum(x0 * x0, axis=1, keepdims=True)
    t = jnp.dot(x0, p_ref[...], preferred_element_type=jnp.float32)
    p0 = t[:, 0:1]
    p1 = t[:, 1:2]
    q0 = t[:, 2:3]
    q1 = t[:, 3:4]
    r0 = t[:, 4:5]
    s0 = a
    s1 = s0 * p0 + q0 + a
    s2 = s1 * p1 + q1 + s1
    uc = uc_ref[...]
    cross = (r0 + s0 * uc[:, 0:1] + s1 * uc[:, 1:2] + s2 * uc[:, 2:3]
             + uc[:, 3:4])
    h = jnp.dot(x0, w1_ref[...], preferred_element_type=jnp.float32)
    h = jnp.maximum(h + b1_ref[...], 0.0)
    h = jnp.dot(h, w2_ref[...], preferred_element_type=jnp.float32)
    h = jnp.maximum(h + b2_ref[...], 0.0)
    out_ref[...] = (cross
                    + jnp.sum(h * wlb_ref[...], axis=1, keepdims=True)
                    + bl_ref[...])


def _tc_dcn(emb2d, Xv, E, P, uc, W1, b1, W2, b2, wlb, bl2,
            bm=256, interpret=False):
    nblk = B // bm
    full = lambda shape: pl.BlockSpec(shape, lambda i: (0, 0))
    out = pl.pallas_call(
        _tc_body,
        grid=(nblk,),
        in_specs=[
            pl.BlockSpec((bm, FD), lambda i: (i, 0)),
            pl.BlockSpec((bm, F_), lambda i: (i, 0)),
            full((F_, FD)),
            full((FD, 8)),
            full((1, 8)),
            full((FD, H1)),
            full((1, H1)),
            full((H1, H2)),
            full((1, H2)),
            full((1, H2)),
            full((1, 1)),
        ],
        out_specs=pl.BlockSpec((bm, 1), lambda i: (i, 0)),
        out_shape=jax.ShapeDtypeStruct((B, 1), jnp.float32),
        compiler_params=pltpu.CompilerParams(
            dimension_semantics=("arbitrary",),
        ),
        interpret=interpret,
    )(emb2d, Xv, E, P, uc, W1, b1, W2, b2, wlb, bl2)
    return out[:, 0]


def kernel(Xi, Xv, tables, cross_w, cross_b, W1, b1, W2, b2, Wl, bl):
    tables_flat = tables.reshape(F_ * V, D)
    gidx = (Xi[:, :, 0].astype(jnp.int32)
            + (jnp.arange(F_, dtype=jnp.int32) * V)[None, :])
    gidx2d = gidx.reshape(_NW, _NCH, _CH)

    emb = _sc_gather(tables_flat, gidx2d)
    emb2d = emb.reshape(B, FD)

    E = jnp.kron(jnp.eye(F_, dtype=jnp.bfloat16),
                 jnp.ones((1, D), dtype=jnp.bfloat16))
    wlt = Wl[:FD, 0]
    wlb = Wl[FD:, 0][None, :]
    bl2 = bl.reshape(1, 1)
    P = jnp.stack([cross_w[0], cross_w[1], cross_b[0], cross_b[1], wlt,
                   jnp.zeros_like(wlt), jnp.zeros_like(wlt),
                   jnp.zeros_like(wlt)], axis=1)
    u = jnp.sum(cross_w * wlt[None, :], axis=1)
    c = jnp.sum(cross_b * wlt[None, :])
    uc = jnp.concatenate([u, c[None], jnp.zeros((4,), jnp.float32)]
                         ).reshape(1, 8)
    return _tc_dcn(emb2d, Xv, E, P, uc, W1, b1.reshape(1, H1),
                   W2, b2.reshape(1, H2), wlb, bl2)
